# Initial kernel scaffold; baseline (speedup 1.0000x reference)
import jax
import jax.numpy as jnp
from jax.experimental import pallas as pl


def kernel(K, V, cb_k, cb_v, step):
    raise NotImplementedError("write your pallas kernel here")


# speedup vs baseline: 1.9240x; 1.9240x over previous
import functools

import jax
import jax.numpy as jnp
from jax import lax
from jax.experimental import pallas as pl
from jax.experimental.pallas import tpu as pltpu
from jax.experimental.pallas import tpu_sc as plsc

_KC = 1024
_D = 128
_TOK = 2 * 16 * 2048
_T = 512
_NT = _TOK // _T
_NW = 32
_TPW = 2 * _TOK // _NW
_RPW = _TPW // 128


def _dist_body(off, z_ref, cbt_ref, idx_ref, loss_ref):
    z = z_ref[...]
    cbt = cbt_ref[...]
    logits = jnp.dot(z, cbt, preferred_element_type=jnp.float32)
    z2 = jnp.sum(z * z, axis=1, keepdims=True)
    c2 = jnp.sum(cbt * cbt, axis=0, keepdims=True)
    dist = z2 + c2 - 2.0 * logits
    m = jnp.min(dist, axis=1, keepdims=True)
    iota = lax.broadcasted_iota(jnp.int32, (_T, _KC), 1)
    idx = jnp.min(jnp.where(dist <= m, iota, _KC), axis=1, keepdims=True)
    idx_ref[0] = idx + off
    loss_ref[0, 0, 0] = jnp.sum(m)


def _nearest(z_flat, cbt, off):
    idx, loss = pl.pallas_call(
        functools.partial(_dist_body, off),
        grid=(_NT,),
        in_specs=[
            pl.BlockSpec((_T, _D), lambda t: (t, 0)),
            pl.BlockSpec((_D, _KC), lambda t: (0, 0)),
        ],
        out_specs=[
            pl.BlockSpec((1, _T, 1), lambda t: (t, 0, 0)),
            pl.BlockSpec((1, 1, 1), lambda t: (t, 0, 0),
                         memory_space=pltpu.SMEM),
        ],
        out_shape=[
            jax.ShapeDtypeStruct((_NT, _T, 1), jnp.int32),
            jax.ShapeDtypeStruct((_NT, 1, 1), jnp.float32),
        ],
    )(z_flat, cbt)
    return idx.reshape(_TOK), loss


def _gather_count_body(cb_ref, idx_ref, zq_ref, cnt_ref,
                       idx_v, rows_v, cnt_v, sem):
    c = lax.axis_index("c")
    s = lax.axis_index("s")
    wid = s * 2 + c
    pltpu.sync_copy(idx_ref.at[pl.ds(wid * _RPW, _RPW)], idx_v)

    def _zero(i, carry):
        cnt_v[pl.ds(i * 16, 16)] = jnp.zeros((16,), jnp.float32)
        return carry

    lax.fori_loop(0, (2 * _KC) // 16, _zero, 0)

    ones = jnp.ones((16,), jnp.float32)

    def _row(j, carry):
        pltpu.async_copy(cb_ref.at[idx_v.at[j]], rows_v, sem).wait()
        pltpu.sync_copy(rows_v, zq_ref.at[pl.ds(wid * _TPW + j * 128, 128)])

        def _cnt(k, inner):
            iv = idx_v[j, pl.ds(k * 16, 16)]
            plsc.addupdate_scatter(cnt_v, [iv], ones)
            return inner

        lax.fori_loop(0, 8, _cnt, 0)
        return carry

    lax.fori_loop(0, _RPW, _row, 0)
    pltpu.sync_copy(cnt_v, cnt_ref.at[wid])


@functools.cache
def _gather_count():
    mesh = plsc.VectorSubcoreMesh(core_axis_name="c", subcore_axis_name="s")
    return pl.kernel(
        _gather_count_body,
        mesh=mesh,
        out_type=[
            jax.ShapeDtypeStruct((2 * _TOK, _D), jnp.float32),
            jax.ShapeDtypeStruct((_NW, 2 * _KC), jnp.float32),
        ],
        scratch_types=[
            pltpu.VMEM((_RPW, 128), jnp.int32),
            pltpu.VMEM((128, _D), jnp.float32),
            pltpu.VMEM((2 * _KC,), jnp.float32),
            pltpu.SemaphoreType.DMA,
        ],
        compiler_params=pltpu.CompilerParams(needs_layout_passes=False),
    )


def kernel(K, V, cb_k, cb_v, step):
    zk = K.reshape(_TOK, _D)
    zv = V.reshape(_TOK, _D)
    idx_k, loss_k = _nearest(zk, cb_k.T, 0)
    idx_vv, loss_v = _nearest(zv, cb_v.T, _KC)
    idx_all = jnp.concatenate([idx_k, idx_vv]).reshape(2 * _TOK // 128, 128)
    cb_cat = jnp.concatenate([cb_k, cb_v], axis=0)
    zq, cnt = _gather_count()(cb_cat, idx_all)
    K_mix = zq[:_TOK].reshape(K.shape)
    V_mix = zq[_TOK:].reshape(V.shape)
    denom = float(_TOK * _D)
    lk = jnp.sum(loss_k) / denom
    lv = jnp.sum(loss_v) / denom
    counts = jnp.sum(cnt, axis=0)
    usage_k = counts[:_KC] / float(_TOK)
    usage_v = counts[_KC:] / float(_TOK)
    return (K_mix, V_mix, 0.25 * lk, 0.25 * lv, 0.25 * lk, 0.25 * lv,
            usage_k, usage_v)


# speedup vs baseline: 2.1651x; 1.1253x over previous
import functools

import jax
import jax.numpy as jnp
from jax import lax
from jax.experimental import pallas as pl
from jax.experimental.pallas import tpu as pltpu
from jax.experimental.pallas import tpu_sc as plsc

_KC = 1024
_D = 128
_TOK = 2 * 16 * 2048
_T = 512
_NT = _TOK // _T
_NW = 32
_TPW = _TOK // _NW
_RPW = _TPW // 128


def _dist_body(z_ref, cbt_ref, idx_ref, loss_ref):
    z = z_ref[...]
    cbt = cbt_ref[...]
    logits = jnp.dot(z, cbt, preferred_element_type=jnp.float32)
    z2 = jnp.sum(z * z, axis=1, keepdims=True)
    c2 = jnp.sum(cbt * cbt, axis=0, keepdims=True)
    dist = z2 + c2 - 2.0 * logits
    m = jnp.min(dist, axis=1, keepdims=True)
    iota = lax.broadcasted_iota(jnp.int32, (_T, _KC), 1)
    idx_ref[0] = jnp.min(jnp.where(dist <= m, iota, _KC),
                         axis=1, keepdims=True)
    loss_ref[0, 0, 0] = jnp.sum(m)


def _nearest(z_flat, cbt):
    idx, loss = pl.pallas_call(
        _dist_body,
        grid=(_NT,),
        in_specs=[
            pl.BlockSpec((_T, _D), lambda t: (t, 0)),
            pl.BlockSpec((_D, _KC), lambda t: (0, 0)),
        ],
        out_specs=[
            pl.BlockSpec((1, _T, 1), lambda t: (t, 0, 0)),
            pl.BlockSpec((1, 1, 1), lambda t: (t, 0, 0),
                         memory_space=pltpu.SMEM),
        ],
        out_shape=[
            jax.ShapeDtypeStruct((_NT, _T, 1), jnp.int32),
            jax.ShapeDtypeStruct((_NT, 1, 1), jnp.float32),
        ],
    )(z_flat, cbt)
    return idx.reshape(_TOK // 128, 128), loss


def _gather_count_body(cb_ref, idx_ref, zq_ref, cnt_ref,
                       idx_v, rows0_v, rows1_v, cnt_v, sem0, sem1):
    c = lax.axis_index("c")
    s = lax.axis_index("s")
    wid = s * 2 + c
    base = wid * _TPW
    pltpu.sync_copy(idx_ref.at[pl.ds(wid * _RPW, _RPW)], idx_v)

    def _zero(i, carry):
        cnt_v[pl.ds(i * 16, 16)] = jnp.zeros((16,), jnp.float32)
        return carry

    lax.fori_loop(0, _KC // 16, _zero, 0)

    ones = jnp.ones((16,), jnp.float32)

    def _count_row(j):
        def _cnt(k, inner):
            iv = idx_v[j, pl.ds(k * 16, 16)]
            plsc.addupdate_scatter(cnt_v, [iv], ones)
            return inner

        lax.fori_loop(0, 8, _cnt, 0)

    def _row_pair(g, carry):
        j0 = 2 * g
        j1 = 2 * g + 1
        cp0 = pltpu.async_copy(cb_ref.at[idx_v.at[j0]], rows0_v, sem0)
        cp1 = pltpu.async_copy(cb_ref.at[idx_v.at[j1]], rows1_v, sem1)
        cp0.wait()
        pltpu.sync_copy(rows0_v, zq_ref.at[pl.ds(base + j0 * 128, 128)])
        cp1.wait()
        pltpu.sync_copy(rows1_v, zq_ref.at[pl.ds(base + j1 * 128, 128)])
        _count_row(j0)
        _count_row(j1)
        return carry

    lax.fori_loop(0, _RPW // 2, _row_pair, 0)
    pltpu.sync_copy(cnt_v, cnt_ref.at[wid])


@functools.cache
def _gather_count():
    mesh = plsc.VectorSubcoreMesh(core_axis_name="c", subcore_axis_name="s")
    return pl.kernel(
        _gather_count_body,
        mesh=mesh,
        out_type=[
            jax.ShapeDtypeStruct((_TOK, _D), jnp.float32),
            jax.ShapeDtypeStruct((_NW, _KC), jnp.float32),
        ],
        scratch_types=[
            pltpu.VMEM((_RPW, 128), jnp.int32),
            pltpu.VMEM((128, _D), jnp.float32),
            pltpu.VMEM((128, _D), jnp.float32),
            pltpu.VMEM((_KC,), jnp.float32),
            pltpu.SemaphoreType.DMA,
            pltpu.SemaphoreType.DMA,
        ],
        compiler_params=pltpu.CompilerParams(needs_layout_passes=False),
    )


def kernel(K, V, cb_k, cb_v, step):
    zk = K.reshape(_TOK, _D)
    zv = V.reshape(_TOK, _D)
    idx_k, loss_k = _nearest(zk, cb_k.T)
    zq_k, cnt_k = _gather_count()(cb_k, idx_k)
    idx_vv, loss_v = _nearest(zv, cb_v.T)
    zq_v, cnt_v = _gather_count()(cb_v, idx_vv)
    K_mix = zq_k.reshape(K.shape)
    V_mix = zq_v.reshape(V.shape)
    denom = float(_TOK * _D)
    lk = jnp.sum(loss_k) / denom
    lv = jnp.sum(loss_v) / denom
    usage_k = jnp.sum(cnt_k, axis=0) / float(_TOK)
    usage_v = jnp.sum(cnt_v, axis=0) / float(_TOK)
    return (K_mix, V_mix, 0.25 * lk, 0.25 * lv, 0.25 * lk, 0.25 * lv,
            usage_k, usage_v)
